# baseline (speedup 1.0000x reference)
import jax
import jax.numpy as jnp
from jax import lax
from jax.experimental import pallas as pl
from jax.experimental.pallas import tpu as pltpu
from jax.experimental.pallas import tpu_sc as plsc

B, L, H, M = 16, 2048, 256, 8192
CH = 64
NCHB = M // CH
WMAX = M // 2
NLANE = 16
RING = 4
NW = 32


def _sc_body(xf, pd, sd, sl, out, tl16,
             pd_v, sd_v, sl_v, tl_v, trow_v, a_ref, g_ref, bufs, zbuf,
             gsems, osems, zsem):
    cid = lax.axis_index("c")
    sid = lax.axis_index("s")
    w = 2 * sid + cid
    lane = lax.iota(jnp.int32, NLANE)

    pltpu.sync_copy(sl, sl_v)
    src_len_own = jnp.sum(jnp.where(lane == sid, sl_v[...], 0))

    zero16 = jnp.zeros((NLANE,), jnp.int32)
    zf16 = jnp.zeros((NLANE,), jnp.float32)

    with jax.named_scope("p0_init"):
        def zinit_body(i, _):
            zbuf[i // (H // NLANE), pl.ds((i % (H // NLANE)) * NLANE, NLANE)] = zf16
            return 0
        lax.fori_loop(0, CH * H // NLANE, zinit_body, 0)

    with jax.named_scope("p1_prepass"):
        def prebatch(bb, _):
            pltpu.sync_copy(pd.at[bb], pd_v)
            pltpu.sync_copy(sd.at[bb], sd_v)
            src_len_bb = jnp.sum(jnp.where(lane == bb, sl_v[...], 0))

            def pre_body(i, acc):
                lvec = i * NLANE + lane
                valid = lvec < src_len_bb
                prr = jnp.maximum(
                    jnp.where(valid, pd_v[pl.ds(i * NLANE, NLANE)], 0), 1)
                srr = jnp.where(valid, sd_v[pl.ds(i * NLANE, NLANE)], 0)
                return acc + prr + srr
            acc = lax.fori_loop(0, L // NLANE, pre_body, zero16)
            tgt_bb = jnp.sum(acc)
            trow_v[...] = jnp.where(lane == bb,
                                    jnp.full((NLANE,), tgt_bb, jnp.int32),
                                    trow_v[...])

            @pl.when((bb == sid) & (cid == 0))
            def _():
                tl_v[...] = jnp.full((NLANE,), tgt_bb, jnp.int32)
                pltpu.sync_copy(tl_v, tl16.at[sid])
            return 0
        lax.fori_loop(0, B, prebatch, 0)

        def vc_of(bb):
            tgt_bb = jnp.max(jnp.where(lane == bb, trow_v[...], 0))
            nvt = jnp.clip(tgt_bb, 0, M)
            return nvt, (nvt + CH - 1) // CH

        def tot_body(bb, carry):
            _, vc = vc_of(bb)
            return carry[0] + vc, carry[1] + (NCHB - vc)
        tvc, tzc = lax.fori_loop(0, B, tot_body, (0, 0))

        vcp = (tvc + NW - 1) // NW
        gv0 = jnp.minimum(w * vcp, tvc)
        gv1 = jnp.minimum(gv0 + vcp, tvc)
        zcp = (tzc + NW - 1) // NW
        gz0 = jnp.minimum(w * zcp, tzc)
        gz1 = jnp.minimum(gz0 + zcp, tzc)

    def gather(c, k):
        pltpu.async_copy(xf.at[g_ref.at[c]], bufs[k], gsems[k])

    def gwait(c, k):
        pltpu.make_async_copy(xf.at[g_ref.at[c]], bufs[k], gsems[k]).wait()

    def process_valid(bb, nvt, lo, ncl, owns_bnd):
        obase = bb * M + lo

        def owait(k):
            pltpu.make_async_copy(bufs[k], out.at[pl.ds(obase, CH)],
                                  osems[k]).wait()

        pltpu.sync_copy(pd.at[bb], pd_v)
        pltpu.sync_copy(sd.at[bb], sd_v)
        src_len = jnp.sum(jnp.where(lane == bb, sl_v[...], 0))
        wlen = ncl * CH

        def az_body(i, _):
            a_ref[pl.ds(i * NLANE, NLANE)] = zero16
            return 0
        lax.fori_loop(0, ncl * (CH // NLANE), az_body, 0)

        def scan_body(i, carry):
            cp, cs, c0 = carry
            lvec = i * NLANE + lane
            valid = lvec < src_len
            prr = jnp.maximum(jnp.where(valid, pd_v[pl.ds(i * NLANE, NLANE)], 0), 1)
            srr = jnp.where(valid, sd_v[pl.ds(i * NLANE, NLANE)], 0)
            P = plsc.cumsum(prr) + cp
            S = plsc.cumsum(srr) + cs
            Pe = P - prr
            Se = S - srr
            key_p = 2 * lvec
            key_s = key_p + 1
            st_p = Pe + Se
            st_s = P + Se
            lp = st_p - lo
            ls = st_s - lo
            mask_p = (lp >= 0) & (lp < wlen)
            mask_s = (ls >= 0) & (ls < wlen) & (srr > 0)
            plsc.store_scatter(a_ref, [jnp.clip(lp, 0, WMAX - 1)], key_p,
                               mask=mask_p)
            plsc.store_scatter(a_ref, [jnp.clip(ls, 0, WMAX - 1)], key_s,
                               mask=mask_s)
            c0 = jnp.maximum(c0, jnp.max(jnp.where(st_p < lo, key_p, 0)))
            c0 = jnp.maximum(c0, jnp.max(jnp.where((st_s < lo) & (srr > 0),
                                                   key_s, 0)))
            return cp + jnp.sum(prr), cs + jnp.sum(srr), c0

        _, _, c0 = lax.fori_loop(0, L // NLANE, scan_body, (0, 0, 0))

        def fill_body(i, cm):
            seg = jnp.maximum(plsc.cummax(a_ref[pl.ds(i * NLANE, NLANE)]), cm)
            tvec = lo + i * NLANE + lane
            row = jnp.where((seg & 1) == 1, 0, seg >> 1)
            gid = bb * L + jnp.where(tvec < nvt, row, 0)
            g_ref[i // (CH // NLANE), pl.ds((i % (CH // NLANE)) * NLANE, NLANE)] = gid
            return jnp.max(seg)
        lax.fori_loop(0, ncl * (CH // NLANE), fill_body, c0)

        gather(0, 0)

        @pl.when(ncl > 1)
        def _():
            gather(1, 1)

        def zero_tail(c, k):
            nvl = nvt - lo

            def zrow(r, _):
                @pl.when(c * CH + r >= nvl)
                def _():
                    for j in range(H // NLANE):
                        bufs[k][r, pl.ds(j * NLANE, NLANE)] = zf16
                return 0
            lax.fori_loop(0, CH, zrow, 0)

        def gbody(i, _):
            for k in range(RING):
                c = RING * i + k

                @pl.when(c < ncl)
                def _():
                    gwait(c, k)

                    @pl.when(owns_bnd & (c == ncl - 1))
                    def _():
                        zero_tail(c, k)
                    pltpu.async_copy(bufs[k], out.at[pl.ds(obase + c * CH, CH)],
                                     osems[k])

                    @pl.when(c + 2 < ncl)
                    def _():
                        k2 = (k + 2) % RING

                        @pl.when(c >= 2)
                        def _():
                            owait(k2)
                        gather(c + 2, k2)
            return 0
        lax.fori_loop(0, (ncl + RING - 1) // RING, gbody, 0)

        for k in range(RING):
            @pl.when(ncl > k)
            def _():
                owait(k)

    def batch_body(bb, carry):
        vpfx, zpfx = carry
        nvt, vc = vc_of(bb)
        a0 = jnp.maximum(gv0, vpfx)
        a1 = jnp.minimum(gv1, vpfx + vc)

        @pl.when(a1 > a0)
        def _():
            process_valid(bb, nvt, (a0 - vpfx) * CH, a1 - a0,
                          a1 == vpfx + vc)

        zc = NCHB - vc
        b0 = jnp.maximum(gz0, zpfx)
        b1 = jnp.minimum(gz1, zpfx + zc)

        @pl.when(b1 > b0)
        def _():
            def zfire(c, _):
                pltpu.async_copy(zbuf, out.at[pl.ds(bb * M + c * CH, CH)], zsem)
                return 0
            lax.fori_loop(vc + (b0 - zpfx), vc + (b1 - zpfx), zfire, 0)

        return vpfx + vc, zpfx + zc

    with jax.named_scope("p3_batches"):
        lax.fori_loop(0, B, batch_body, (0, 0))

    with jax.named_scope("p4_drain"):
        def zdrain(j, _):
            pltpu.make_async_copy(zbuf, out.at[pl.ds(0, CH)], zsem).wait()
            return 0
        lax.fori_loop(0, gz1 - gz0, zdrain, 0)


@jax.jit
def _run(xf, pd, sd, sl):
    mesh = plsc.VectorSubcoreMesh(core_axis_name="c", subcore_axis_name="s")
    f = pl.kernel(
        _sc_body,
        out_type=(
            jax.ShapeDtypeStruct((B * M, H), jnp.float32),
            jax.ShapeDtypeStruct((16, 16), jnp.int32),
        ),
        mesh=mesh,
        scratch_types=[
            pltpu.VMEM((L,), jnp.int32),
            pltpu.VMEM((L,), jnp.int32),
            pltpu.VMEM((16,), jnp.int32),
            pltpu.VMEM((16,), jnp.int32),
            pltpu.VMEM((16,), jnp.int32),
            pltpu.VMEM((WMAX,), jnp.int32),
            pltpu.VMEM((WMAX // CH, CH), jnp.int32),
            [pltpu.VMEM((CH, H), jnp.float32) for _ in range(RING)],
            pltpu.VMEM((CH, H), jnp.float32),
            [pltpu.SemaphoreType.DMA for _ in range(RING)],
            [pltpu.SemaphoreType.DMA for _ in range(RING)],
            pltpu.SemaphoreType.DMA,
        ],
        compiler_params=pltpu.CompilerParams(needs_layout_passes=False),
    )
    return f(xf, pd, sd, sl)


def kernel(x, phone_duration, sil_duration, src_lens, max_len):
    out_flat, tl16 = _run(x.reshape(B * L, H), phone_duration,
                          sil_duration, src_lens)
    return out_flat.reshape(B, M, H), tl16[:, 0]
